# baseline (speedup 1.0000x reference)
import functools

import jax
import jax.numpy as jnp
from jax.experimental import pallas as pl
from jax.experimental.pallas import tpu as pltpu
from jax.experimental.pallas import tpu_sc as plsc

D_MODEL = 64
SCALE = 8.0
WINDOW = 128
LANES = 16


def _emb_kernel(b_total, table_hbm, idx_hbm, out_hbm):
    def body(i_vmem, o_vmem):
        pltpu.sync_copy(table_hbm.at[i_vmem.at[0]], o_vmem)

        @pl.loop(0, WINDOW)
        def _(r):
            for c in range(D_MODEL // LANES):
                slc = (pl.ds(r, 1), pl.ds(c * LANES, LANES))
                o_vmem.at[*slc][...] = o_vmem.at[*slc][...] * SCALE

    pltpu.emit_pipeline(
        body,
        grid=(b_total // WINDOW,),
        in_specs=[pl.BlockSpec((1, WINDOW), index_map=lambda i: (0, i))],
        out_specs=[pl.BlockSpec((WINDOW, D_MODEL), index_map=lambda i: (i, 0))],
        core_axis_name=("core", "subcore"),
        dimension_semantics=(pltpu.PARALLEL,),
    )(idx_hbm, out_hbm)


def kernel(x, table):
    b_total = x.shape[0] * x.shape[1]
    idx = x.reshape(1, b_total)
    mesh = plsc.VectorSubcoreMesh(core_axis_name="core",
                                  subcore_axis_name="subcore")

    @functools.partial(
        pl.kernel,
        out_type=jax.ShapeDtypeStruct((b_total, D_MODEL), jnp.float32),
        mesh=mesh,
        compiler_params=pltpu.CompilerParams(use_tc_tiling_on_sc=False),
    )
    def run(table_ref, idx_ref, out_ref):
        _emb_kernel(b_total, table_ref, idx_ref, out_ref)

    out = run(table, idx)
    return out.reshape(x.shape[0], x.shape[1], D_MODEL)


# speedup vs baseline: 1.4890x; 1.4890x over previous
import functools

import jax
import jax.numpy as jnp
from jax import lax
from jax.experimental import pallas as pl
from jax.experimental.pallas import tpu as pltpu
from jax.experimental.pallas import tpu_sc as plsc

D_MODEL = 64
SCALE = 8.0
W = 128
LANES = 16
NBUF = 4
NW = 32


def _worker_body(nch, wid, table_hbm, idx_hbm, out_hbm,
                 idx_bufs, row_bufs, idx_sems, g_sems, s_sems):
    def idx_start(g, b):
        pltpu.async_copy(idx_hbm.at[wid * nch + g], idx_bufs[b], idx_sems[b])

    def idx_wait(g, b):
        pltpu.make_async_copy(idx_hbm.at[wid * nch + g], idx_bufs[b],
                              idx_sems[b]).wait()

    def gather_start(b):
        pltpu.async_copy(table_hbm.at[idx_bufs[b]], row_bufs[b], g_sems[b])

    def gather_wait(b):
        pltpu.make_async_copy(table_hbm.at[idx_bufs[b]], row_bufs[b],
                              g_sems[b]).wait()

    def out_slice(g):
        return out_hbm.at[pl.ds((wid * nch + g) * W, W)]

    def scatter_start(g, b):
        pltpu.async_copy(row_bufs[b], out_slice(g), s_sems[b])

    def scatter_wait(g, b):
        pltpu.make_async_copy(row_bufs[b], out_slice(g), s_sems[b]).wait()

    def scale(b):
        rows = row_bufs[b]

        @pl.loop(0, W, unroll=4)
        def _(r):
            for c in range(D_MODEL // LANES):
                slc = (pl.ds(r, 1), pl.ds(c * LANES, LANES))
                rows.at[*slc][...] = rows.at[*slc][...] * SCALE

    idx_start(0, 0)
    idx_start(1, 1)
    idx_start(2, 2)
    idx_wait(0, 0)
    gather_start(0)
    idx_wait(1, 1)
    gather_start(1)

    def step(g, b):
        @pl.when(g + 3 < nch)
        def _():
            idx_start(g + 3, (b + 3) % NBUF)

        @pl.when(g + 2 < nch)
        def _():
            @pl.when(g >= 2)
            def _():
                scatter_wait(g - 2, (b + 2) % NBUF)
            idx_wait(g + 2, (b + 2) % NBUF)
            gather_start((b + 2) % NBUF)

        gather_wait(b)
        scale(b)
        scatter_start(g, b)

    @pl.loop(0, nch // NBUF)
    def _(i):
        g0 = i * NBUF
        for k in range(NBUF):
            step(g0 + k, k)

    for g in (nch - 4, nch - 3, nch - 2, nch - 1):
        scatter_wait(g, g % NBUF)


def kernel(x, table):
    b_total = x.shape[0] * x.shape[1]
    n_windows = b_total // W
    nch = n_windows // NW
    idx = x.reshape(n_windows, W)
    mesh = plsc.VectorSubcoreMesh(core_axis_name="core",
                                  subcore_axis_name="subcore")

    scratch = (
        [pltpu.VMEM((W,), jnp.int32) for _ in range(NBUF)]
        + [pltpu.VMEM((W, D_MODEL), jnp.float32) for _ in range(NBUF)]
        + [pltpu.SemaphoreType.DMA for _ in range(3 * NBUF)]
    )

    @functools.partial(
        pl.kernel,
        out_type=jax.ShapeDtypeStruct((b_total, D_MODEL), jnp.float32),
        mesh=mesh,
        scratch_types=scratch,
        compiler_params=pltpu.CompilerParams(use_tc_tiling_on_sc=False),
    )
    def run(table_ref, idx_ref, out_ref, *scratch_refs):
        idx_bufs = scratch_refs[0:NBUF]
        row_bufs = scratch_refs[NBUF:2 * NBUF]
        idx_sems = scratch_refs[2 * NBUF:3 * NBUF]
        g_sems = scratch_refs[3 * NBUF:4 * NBUF]
        s_sems = scratch_refs[4 * NBUF:5 * NBUF]
        wid = lax.axis_index("core") * 16 + lax.axis_index("subcore")
        _worker_body(nch, wid, table_ref, idx_ref, out_ref,
                     idx_bufs, row_bufs, idx_sems, g_sems, s_sems)

    out = run(table, idx)
    return out.reshape(x.shape[0], x.shape[1], D_MODEL)
